# baseline (speedup 1.0000x reference)
import functools

import jax
import jax.numpy as jnp
from jax import lax
from jax.experimental import pallas as pl
from jax.experimental.pallas import tpu as pltpu
from jax.experimental.pallas import tpu_sc as plsc

N = 10000
E = 160000
D = 256
DH = 128
NC, NS, L = 2, 16, 16

CH = 64
NCHUNK = 158
EPAD = NS * NCHUNK * CH
PSHIFT = 15
PMASK = (1 << PSHIFT) - 1
NPAD = 10240
ROWS_PER_TILE = NPAD // NS
DRAIN = CH
DEG_PER_TILE = NPAD // NS

def _deg_body(rows_hbm, degp_hbm, rowv, onesv, dbuf, deg_sh):
    c = lax.axis_index("c")
    s = lax.axis_index("s")
    for i in range(DEG_PER_TILE // L):
        dbuf[pl.ds(i * L, L)] = jnp.zeros((L,), jnp.float32)
    for i in range(CH // L):
        onesv[pl.ds(i * L, L)] = jnp.ones((L,), jnp.float32)
    pltpu.sync_copy(dbuf, deg_sh.at[pl.ds(s * DEG_PER_TILE, DEG_PER_TILE)])
    pltpu.sync_copy(rows_hbm.at[s], rowv)
    plsc.subcore_barrier()

    lo = (NCHUNK // 2) * c
    hi = lo + NCHUNK // 2

    def body(j, carry):
        pltpu.sync_copy(onesv, deg_sh.at[rowv.at[j]], add=True)
        return carry

    lax.fori_loop(lo, hi, body, 0)
    plsc.subcore_barrier()
    pltpu.sync_copy(deg_sh.at[pl.ds(s * DEG_PER_TILE, DEG_PER_TILE)], dbuf)
    pltpu.sync_copy(dbuf, degp_hbm.at[c, pl.ds(s * DEG_PER_TILE, DEG_PER_TILE)])


def _mm_body(x_ref, w_ref, degp_ref, xws_ref, dis_ref):
    deg = degp_ref[0] + degp_ref[1]
    dis = jnp.where(deg > 0.0, lax.rsqrt(deg), 0.0)
    xw = jnp.dot(x_ref[...], w_ref[...], preferred_element_type=jnp.float32)
    xws = xw * dis
    xws_ref[0, :, :] = xws[:, :DH]
    xws_ref[1, :, :] = xws[:, DH:]
    dis_ref[...] = dis


_MM_B = 2000

_mm_call = pl.pallas_call(
    _mm_body,
    grid=(N // _MM_B,),
    in_specs=[
        pl.BlockSpec((_MM_B, D), lambda i: (i, 0)),
        pl.BlockSpec((D, D), lambda i: (0, 0)),
        pl.BlockSpec((NC, _MM_B, 1), lambda i: (0, i, 0)),
    ],
    out_specs=[
        pl.BlockSpec((NC, _MM_B, DH), lambda i: (0, i, 0)),
        pl.BlockSpec((_MM_B, 1), lambda i: (i, 0)),
    ],
    out_shape=[
        jax.ShapeDtypeStruct((NC, N, DH), jnp.float32),
        jax.ShapeDtypeStruct((N, 1), jnp.float32),
    ],
)


def _agg_body(xws_hbm, packed_hbm, acc_hbm,
              pidv, ridx0, cidx0, ridx1, cidx1, gbuf0, gbuf1, acc_sh,
              gsem0, gsem1, ssem0, ssem1):
    c = lax.axis_index("c")
    s = lax.axis_index("s")
    ridx = (ridx0, ridx1)
    cidx = (cidx0, cidx1)
    gbuf = (gbuf0, gbuf1)
    gsem = (gsem0, gsem1)
    ssem = (ssem0, ssem1)

    pltpu.async_copy(packed_hbm.at[c, s], pidv, gsem0)

    def zrow(v, carry):
        for gg in range(DH // L):
            gbuf0[v, pl.ds(gg * L, L)] = jnp.zeros((L,), jnp.float32)
        return carry

    lax.fori_loop(0, DRAIN, zrow, 0)
    for t in range(ROWS_PER_TILE // DRAIN):
        pltpu.async_copy(gbuf0, acc_sh.at[pl.ds(s * ROWS_PER_TILE + t * DRAIN,
                                                DRAIN)], ssem0)
    for t in range(ROWS_PER_TILE // DRAIN):
        pltpu.make_async_copy(gbuf0,
                              acc_sh.at[pl.ds(s * ROWS_PER_TILE + t * DRAIN,
                                              DRAIN)], ssem0).wait()
    pltpu.make_async_copy(packed_hbm.at[c, s], pidv, gsem0).wait()
    plsc.subcore_barrier()

    def unpack(j, p):
        for i in range(CH // L):
            v = pidv[j, pl.ds(i * L, L)]
            ridx[p][0, pl.ds(i * L, L)] = lax.shift_right_logical(v, PSHIFT)
            cidx[p][0, pl.ds(i * L, L)] = v & PMASK

    def start_gather(p):
        pltpu.async_copy(xws_hbm.at[cidx[p].at[0]], gbuf[p], gsem[p])

    def wait_gather(p):
        pltpu.make_async_copy(xws_hbm.at[cidx[p].at[0]], gbuf[p],
                              gsem[p]).wait()

    def start_scatter(p):
        pltpu.async_copy(gbuf[p], acc_sh.at[ridx[p].at[0]], ssem[p],
                         add=True)

    def wait_scatter(p):
        pltpu.make_async_copy(xws_hbm.at[pl.ds(0, CH)], gbuf[p],
                              ssem[p]).wait()

    unpack(0, 0)
    start_gather(0)
    unpack(1, 1)
    start_gather(1)

    @pl.loop(0, NCHUNK - 2, step=2)
    def _chunks(j):
        wait_gather(0)
        start_scatter(0)
        wait_scatter(0)
        unpack(j + 2, 0)
        start_gather(0)
        wait_gather(1)
        start_scatter(1)
        wait_scatter(1)
        unpack(j + 3, 1)
        start_gather(1)

    wait_gather(0)
    start_scatter(0)
    wait_scatter(0)
    wait_gather(1)
    start_scatter(1)
    wait_scatter(1)
    plsc.subcore_barrier()

    for t in range(ROWS_PER_TILE // DRAIN):
        r0 = s * ROWS_PER_TILE + t * DRAIN
        pltpu.async_copy(acc_sh.at[pl.ds(r0, DRAIN)],
                         acc_hbm.at[c, pl.ds(r0, DRAIN)], gsem0)
    for t in range(ROWS_PER_TILE // DRAIN):
        r0 = s * ROWS_PER_TILE + t * DRAIN
        pltpu.make_async_copy(acc_sh.at[pl.ds(r0, DRAIN)],
                              acc_hbm.at[c, pl.ds(r0, DRAIN)], gsem0).wait()


def _post_body(acc_ref, dis_ref, out_ref):
    dis = dis_ref[...]
    a0 = acc_ref[0].astype(jnp.float32)
    a1 = acc_ref[1].astype(jnp.float32)
    out_ref[...] = jnp.concatenate([a0 * dis, a1 * dis], axis=1)


_post_call = pl.pallas_call(
    _post_body,
    grid=(N // _MM_B,),
    in_specs=[
        pl.BlockSpec((NC, _MM_B, DH), lambda i: (0, i, 0)),
        pl.BlockSpec((_MM_B, 1), lambda i: (i, 0)),
    ],
    out_specs=pl.BlockSpec((_MM_B, D), lambda i: (i, 0)),
    out_shape=jax.ShapeDtypeStruct((N, D), jnp.float32),
)


@functools.cache
def _sc_kernels():
    mesh = plsc.VectorSubcoreMesh(
        core_axis_name="c", subcore_axis_name="s",
        num_cores=NC, num_subcores=NS)
    deg_kernel = pl.kernel(
        _deg_body,
        mesh=mesh,
        out_type=jax.ShapeDtypeStruct((NC, NPAD), jnp.float32),
        scratch_types=[
            pltpu.VMEM((NCHUNK, CH), jnp.int32),
            pltpu.VMEM((CH,), jnp.float32),
            pltpu.VMEM((DEG_PER_TILE,), jnp.float32),
            pltpu.VMEM_SHARED((NPAD,), jnp.float32),
        ],
    )
    agg_kernel = pl.kernel(
        _agg_body,
        mesh=mesh,
        out_type=jax.ShapeDtypeStruct((NC, NPAD, DH), jnp.float32),
        scratch_types=[
            pltpu.VMEM((NCHUNK, CH), jnp.int32),
            pltpu.VMEM((8, CH), jnp.int32),
            pltpu.VMEM((8, CH), jnp.int32),
            pltpu.VMEM((8, CH), jnp.int32),
            pltpu.VMEM((8, CH), jnp.int32),
            pltpu.VMEM((CH, DH), jnp.float32),
            pltpu.VMEM((CH, DH), jnp.float32),
            pltpu.VMEM_SHARED((NPAD, DH), jnp.float32),
            pltpu.SemaphoreType.DMA,
            pltpu.SemaphoreType.DMA,
            pltpu.SemaphoreType.DMA,
            pltpu.SemaphoreType.DMA,
        ],
    )
    return deg_kernel, agg_kernel


def kernel(x, edge_index, W):
    ei = edge_index.astype(jnp.int32)
    row = jnp.concatenate([ei[0], jnp.full((EPAD - E,), N, jnp.int32)])
    col = jnp.concatenate([ei[1], jnp.zeros((EPAD - E,), jnp.int32)])
    rows3 = row.reshape(NS, NCHUNK, CH)
    packed = ((row << PSHIFT) | jnp.stack([col, col + N])
              ).reshape(NC, NS, NCHUNK, CH)

    deg_kernel, agg_kernel = _sc_kernels()
    degp = deg_kernel(rows3)
    degp3 = degp[:, :N].reshape(NC, N, 1)
    xws, dis = _mm_call(x, W, degp3)
    acc = agg_kernel(xws.reshape(NC * N, DH), packed)
    return _post_call(acc, dis)


# speedup vs baseline: 1.1382x; 1.1382x over previous
import functools

import jax
import jax.numpy as jnp
from jax import lax
from jax.experimental import pallas as pl
from jax.experimental.pallas import tpu as pltpu
from jax.experimental.pallas import tpu_sc as plsc

N = 10000
E = 160000
D = 256
DH = 128
NC, NS, L = 2, 16, 16

CH = 80
NCHUNK = 126
EPAD = NS * NCHUNK * CH
PSHIFT = 15
PMASK = (1 << PSHIFT) - 1
NPAD = 10240
ROWS_PER_TILE = NPAD // NS
DRAIN = CH
DEG_PER_TILE = NPAD // NS

def _deg_body(rows_hbm, degp_hbm, rowv, onesv, dbuf, deg_sh):
    c = lax.axis_index("c")
    s = lax.axis_index("s")
    for i in range(DEG_PER_TILE // L):
        dbuf[pl.ds(i * L, L)] = jnp.zeros((L,), jnp.float32)
    for i in range(CH // L):
        onesv[pl.ds(i * L, L)] = jnp.ones((L,), jnp.float32)
    pltpu.sync_copy(dbuf, deg_sh.at[pl.ds(s * DEG_PER_TILE, DEG_PER_TILE)])
    pltpu.sync_copy(rows_hbm.at[s], rowv)
    plsc.subcore_barrier()

    lo = (NCHUNK // 2) * c
    hi = lo + NCHUNK // 2

    def body(j, carry):
        pltpu.sync_copy(onesv, deg_sh.at[rowv.at[j]], add=True)
        return carry

    lax.fori_loop(lo, hi, body, 0)
    plsc.subcore_barrier()
    pltpu.sync_copy(deg_sh.at[pl.ds(s * DEG_PER_TILE, DEG_PER_TILE)], dbuf)
    pltpu.sync_copy(dbuf, degp_hbm.at[c, pl.ds(s * DEG_PER_TILE, DEG_PER_TILE)])


def _mm_body(x_ref, w_ref, degp_ref, xws_ref, dis_ref):
    deg = degp_ref[0] + degp_ref[1]
    dis = jnp.where(deg > 0.0, lax.rsqrt(deg), 0.0)
    xw = jnp.dot(x_ref[...], w_ref[...], preferred_element_type=jnp.float32)
    xws = xw * dis
    xws_ref[0, :, :] = xws[:, :DH]
    xws_ref[1, :, :] = xws[:, DH:]
    dis_ref[...] = dis


_MM_B = 2000

_mm_call = pl.pallas_call(
    _mm_body,
    grid=(N // _MM_B,),
    in_specs=[
        pl.BlockSpec((_MM_B, D), lambda i: (i, 0)),
        pl.BlockSpec((D, D), lambda i: (0, 0)),
        pl.BlockSpec((NC, _MM_B, 1), lambda i: (0, i, 0)),
    ],
    out_specs=[
        pl.BlockSpec((NC, _MM_B, DH), lambda i: (0, i, 0)),
        pl.BlockSpec((_MM_B, 1), lambda i: (i, 0)),
    ],
    out_shape=[
        jax.ShapeDtypeStruct((NC, N, DH), jnp.float32),
        jax.ShapeDtypeStruct((N, 1), jnp.float32),
    ],
)


def _agg_body(xws_hbm, packed_hbm, acc_hbm,
              pidv, ridx0, cidx0, ridx1, cidx1, gbuf0, gbuf1, acc_sh,
              gsem0, gsem1, ssem0, ssem1):
    c = lax.axis_index("c")
    s = lax.axis_index("s")
    ridx = (ridx0, ridx1)
    cidx = (cidx0, cidx1)
    gbuf = (gbuf0, gbuf1)
    gsem = (gsem0, gsem1)
    ssem = (ssem0, ssem1)

    pltpu.async_copy(packed_hbm.at[c, s], pidv, gsem0)

    def zrow(v, carry):
        for gg in range(DH // L):
            gbuf0[v, pl.ds(gg * L, L)] = jnp.zeros((L,), jnp.float32)
        return carry

    lax.fori_loop(0, DRAIN, zrow, 0)
    for t in range(ROWS_PER_TILE // DRAIN):
        pltpu.async_copy(gbuf0, acc_sh.at[pl.ds(s * ROWS_PER_TILE + t * DRAIN,
                                                DRAIN)], ssem0)
    for t in range(ROWS_PER_TILE // DRAIN):
        pltpu.make_async_copy(gbuf0,
                              acc_sh.at[pl.ds(s * ROWS_PER_TILE + t * DRAIN,
                                              DRAIN)], ssem0).wait()
    pltpu.make_async_copy(packed_hbm.at[c, s], pidv, gsem0).wait()
    plsc.subcore_barrier()

    def unpack(j, p):
        for i in range(CH // L):
            v = pidv[j, pl.ds(i * L, L)]
            ridx[p][0, pl.ds(i * L, L)] = lax.shift_right_logical(v, PSHIFT)
            cidx[p][0, pl.ds(i * L, L)] = v & PMASK

    def start_gather(p):
        pltpu.async_copy(xws_hbm.at[cidx[p].at[0]], gbuf[p], gsem[p])

    def wait_gather(p):
        pltpu.make_async_copy(xws_hbm.at[cidx[p].at[0]], gbuf[p],
                              gsem[p]).wait()

    def start_scatter(p):
        pltpu.async_copy(gbuf[p], acc_sh.at[ridx[p].at[0]], ssem[p],
                         add=True)

    def wait_scatter(p):
        pltpu.make_async_copy(xws_hbm.at[pl.ds(0, CH)], gbuf[p],
                              ssem[p]).wait()

    unpack(0, 0)
    start_gather(0)
    unpack(1, 1)
    start_gather(1)

    @pl.loop(0, NCHUNK - 2, step=2)
    def _chunks(j):
        wait_gather(0)
        start_scatter(0)
        wait_scatter(0)
        unpack(j + 2, 0)
        start_gather(0)
        wait_gather(1)
        start_scatter(1)
        wait_scatter(1)
        unpack(j + 3, 1)
        start_gather(1)

    wait_gather(0)
    start_scatter(0)
    wait_scatter(0)
    wait_gather(1)
    start_scatter(1)
    wait_scatter(1)
    plsc.subcore_barrier()

    for t in range(ROWS_PER_TILE // DRAIN):
        r0 = s * ROWS_PER_TILE + t * DRAIN
        pltpu.async_copy(acc_sh.at[pl.ds(r0, DRAIN)],
                         acc_hbm.at[c, pl.ds(r0, DRAIN)], gsem0)
    for t in range(ROWS_PER_TILE // DRAIN):
        r0 = s * ROWS_PER_TILE + t * DRAIN
        pltpu.make_async_copy(acc_sh.at[pl.ds(r0, DRAIN)],
                              acc_hbm.at[c, pl.ds(r0, DRAIN)], gsem0).wait()


def _post_body(acc_ref, dis_ref, out_ref):
    dis = dis_ref[...]
    a0 = acc_ref[0].astype(jnp.float32)
    a1 = acc_ref[1].astype(jnp.float32)
    out_ref[...] = jnp.concatenate([a0 * dis, a1 * dis], axis=1)


_post_call = pl.pallas_call(
    _post_body,
    grid=(N // _MM_B,),
    in_specs=[
        pl.BlockSpec((NC, _MM_B, DH), lambda i: (0, i, 0)),
        pl.BlockSpec((_MM_B, 1), lambda i: (i, 0)),
    ],
    out_specs=pl.BlockSpec((_MM_B, D), lambda i: (i, 0)),
    out_shape=jax.ShapeDtypeStruct((N, D), jnp.float32),
)


@functools.cache
def _sc_kernels():
    mesh = plsc.VectorSubcoreMesh(
        core_axis_name="c", subcore_axis_name="s",
        num_cores=NC, num_subcores=NS)
    deg_kernel = pl.kernel(
        _deg_body,
        mesh=mesh,
        out_type=jax.ShapeDtypeStruct((NC, NPAD), jnp.float32),
        scratch_types=[
            pltpu.VMEM((NCHUNK, CH), jnp.int32),
            pltpu.VMEM((CH,), jnp.float32),
            pltpu.VMEM((DEG_PER_TILE,), jnp.float32),
            pltpu.VMEM_SHARED((NPAD,), jnp.float32),
        ],
    )
    agg_kernel = pl.kernel(
        _agg_body,
        mesh=mesh,
        out_type=jax.ShapeDtypeStruct((NC, NPAD, DH), jnp.float32),
        scratch_types=[
            pltpu.VMEM((NCHUNK, CH), jnp.int32),
            pltpu.VMEM((8, CH), jnp.int32),
            pltpu.VMEM((8, CH), jnp.int32),
            pltpu.VMEM((8, CH), jnp.int32),
            pltpu.VMEM((8, CH), jnp.int32),
            pltpu.VMEM((CH, DH), jnp.float32),
            pltpu.VMEM((CH, DH), jnp.float32),
            pltpu.VMEM_SHARED((NPAD, DH), jnp.float32),
            pltpu.SemaphoreType.DMA,
            pltpu.SemaphoreType.DMA,
            pltpu.SemaphoreType.DMA,
            pltpu.SemaphoreType.DMA,
        ],
    )
    return deg_kernel, agg_kernel


def kernel(x, edge_index, W):
    ei = edge_index.astype(jnp.int32)
    row = jnp.concatenate([ei[0], jnp.full((EPAD - E,), N, jnp.int32)])
    col = jnp.concatenate([ei[1], jnp.zeros((EPAD - E,), jnp.int32)])
    rows3 = row.reshape(NS, NCHUNK, CH)
    packed = ((row << PSHIFT) | jnp.stack([col, col + N])
              ).reshape(NC, NS, NCHUNK, CH)

    deg_kernel, agg_kernel = _sc_kernels()
    degp = deg_kernel(rows3)
    degp3 = degp[:, :N].reshape(NC, N, 1)
    xws, dis = _mm_call(x, W, degp3)
    acc = agg_kernel(xws.reshape(NC * N, DH), packed)
    return _post_call(acc, dis)
